# Initial kernel scaffold; baseline (speedup 1.0000x reference)
import jax
import jax.numpy as jnp
from jax.experimental import pallas as pl


def kernel(x, adj, W1, W2, Wo, bo):
    raise NotImplementedError("write your pallas kernel here")


# speedup vs baseline: 1.7818x; 1.7818x over previous
import jax
import jax.numpy as jnp
from jax.experimental import pallas as pl
from jax.experimental.pallas import tpu as pltpu


def _support_body(x_ref, w1_ref, out_ref):
    out_ref[...] = jnp.dot(x_ref[...], w1_ref[...],
                           preferred_element_type=jnp.float32)


def _main_body(adj_ref, s1_ref, w2_ref, wo_ref, bo_ref, out_ref,
               c_acc, h1_acc):
    i = pl.program_id(0)
    ti = adj_ref.shape[0]
    blk = adj_ref[...]

    h1 = jnp.maximum(
        jnp.dot(blk, s1_ref[...], preferred_element_type=jnp.float32), 0.0)
    h1_acc[pl.ds(i * ti, ti), :] = h1

    colsum = jnp.sum(blk, axis=0, keepdims=True)

    @pl.when(i == 0)
    def _init():
        c_acc[...] = colsum

    @pl.when(i > 0)
    def _accum():
        c_acc[...] = c_acc[...] + colsum

    @pl.when(i == pl.num_programs(0) - 1)
    def _finish():
        pooled = jnp.dot(c_acc[...], h1_acc[...],
                         preferred_element_type=jnp.float32)
        z = jnp.dot(pooled, w2_ref[...],
                    preferred_element_type=jnp.float32)
        o = jnp.dot(z, wo_ref[...],
                    preferred_element_type=jnp.float32) + bo_ref[...]
        out_ref[...] = jax.nn.sigmoid(o)


def kernel(x, adj, W1, W2, Wo, bo):
    n, d_in = x.shape
    h1_dim = W1.shape[1]
    h2_dim = W2.shape[1]

    tx = 1000
    ti = 400

    support1 = pl.pallas_call(
        _support_body,
        grid=(n // tx,),
        in_specs=[
            pl.BlockSpec((tx, d_in), lambda i: (i, 0)),
            pl.BlockSpec((d_in, h1_dim), lambda i: (0, 0)),
        ],
        out_specs=pl.BlockSpec((tx, h1_dim), lambda i: (i, 0)),
        out_shape=jax.ShapeDtypeStruct((n, h1_dim), jnp.float32),
    )(x, W1)

    out = pl.pallas_call(
        _main_body,
        grid=(n // ti,),
        in_specs=[
            pl.BlockSpec((ti, n), lambda i: (i, 0)),
            pl.BlockSpec((n, h1_dim), lambda i: (0, 0)),
            pl.BlockSpec((h1_dim, h2_dim), lambda i: (0, 0)),
            pl.BlockSpec((h2_dim, 1), lambda i: (0, 0)),
            pl.BlockSpec((1, 1), lambda i: (0, 0)),
        ],
        out_specs=pl.BlockSpec((1, 1), lambda i: (0, 0)),
        out_shape=jax.ShapeDtypeStruct((1, 1), jnp.float32),
        scratch_shapes=[
            pltpu.VMEM((1, n), jnp.float32),
            pltpu.VMEM((n, h1_dim), jnp.float32),
        ],
        compiler_params=pltpu.CompilerParams(
            dimension_semantics=("arbitrary",)),
    )(adj, support1, W2, Wo, bo.reshape(1, 1))

    return out.reshape(1)


# speedup vs baseline: 1.8266x; 1.0251x over previous
import jax
import jax.numpy as jnp
from jax.experimental import pallas as pl
from jax.experimental.pallas import tpu as pltpu


def _body(x_ref, adj_ref, w1_ref, w2_ref, wo_ref, bo_ref, out_ref,
          s1, c_acc, h1_acc):
    i = pl.program_id(0)
    nsteps = pl.num_programs(0)
    ti = adj_ref.shape[0]

    @pl.when(i == 0)
    def _prologue():
        s1[...] = jnp.dot(x_ref[...], w1_ref[...],
                          preferred_element_type=jnp.float32)

    @pl.when(i > 0)
    def _stream():
        r = i - 1
        blk = adj_ref[...]
        h1 = jnp.maximum(
            jnp.dot(blk, s1[...], preferred_element_type=jnp.float32), 0.0)
        h1_acc[pl.ds(r * ti, ti), :] = h1
        colsum = jnp.sum(blk, axis=0, keepdims=True)
        c_acc[...] = jnp.where(r == 0, colsum, c_acc[...] + colsum)

    @pl.when(i == nsteps - 1)
    def _epilogue():
        pooled = jnp.dot(c_acc[...], h1_acc[...],
                         preferred_element_type=jnp.float32)
        z = jnp.dot(pooled, w2_ref[...],
                    preferred_element_type=jnp.float32)
        o = jnp.dot(z, wo_ref[...],
                    preferred_element_type=jnp.float32) + bo_ref[...]
        out_ref[...] = jax.nn.sigmoid(o)


def kernel(x, adj, W1, W2, Wo, bo):
    n, d_in = x.shape
    h1_dim = W1.shape[1]
    h2_dim = W2.shape[1]

    ti = 200
    nblk = n // ti

    out = pl.pallas_call(
        _body,
        grid=(nblk + 1,),
        in_specs=[
            pl.BlockSpec((n, d_in), lambda i: (0, 0)),
            pl.BlockSpec((ti, n), lambda i: (jnp.maximum(i - 1, 0), 0)),
            pl.BlockSpec((d_in, h1_dim), lambda i: (0, 0)),
            pl.BlockSpec((h1_dim, h2_dim), lambda i: (0, 0)),
            pl.BlockSpec((h2_dim, 1), lambda i: (0, 0)),
            pl.BlockSpec((1, 1), lambda i: (0, 0)),
        ],
        out_specs=pl.BlockSpec((1, 1), lambda i: (0, 0)),
        out_shape=jax.ShapeDtypeStruct((1, 1), jnp.float32),
        scratch_shapes=[
            pltpu.VMEM((n, h1_dim), jnp.float32),
            pltpu.VMEM((1, n), jnp.float32),
            pltpu.VMEM((n, h1_dim), jnp.float32),
        ],
        compiler_params=pltpu.CompilerParams(
            dimension_semantics=("arbitrary",)),
    )(x, adj, W1, W2, Wo, bo.reshape(1, 1))

    return out.reshape(1)
